# baseline (speedup 1.0000x reference)
import functools
import math

import jax
import jax.numpy as jnp
from jax import lax
from jax.experimental import pallas as pl
from jax.experimental.pallas import tpu as pltpu
from jax.experimental.pallas import tpu_sc as plsc

D_MODEL = 1024
SCALE = math.sqrt(D_MODEL)
LANES = 16
VECS_PER_ROW = D_MODEL // LANES
C = 32
NBUF = 3


@functools.lru_cache(maxsize=None)
def _build_sc_embed(rows_x, cols_x, num_cores, num_subcores):
    B = rows_x * cols_x
    NW = num_cores * num_subcores
    b_per_w = B // NW
    w_per_row = cols_x // b_per_w
    assert w_per_row * b_per_w == cols_x
    C0 = 8
    n_full = (b_per_w - C0) // C
    CT = b_per_w - C0 - (n_full - 1) * C - C
    assert CT == C - C0
    n_groups = n_full // NBUF
    assert n_groups * NBUF == n_full and n_groups >= 2

    def chunk_off(g):
        return 0 if g == 0 else C0 + (g - 1) * C
    mesh = plsc.VectorSubcoreMesh(core_axis_name="c", subcore_axis_name="s")

    @functools.partial(
        pl.kernel,
        mesh=mesh,
        out_type=jax.ShapeDtypeStruct((B, D_MODEL), jnp.float32),
        scratch_types=[
            pltpu.VMEM((b_per_w,), jnp.int32),
            *[pltpu.VMEM((C, D_MODEL), jnp.float32) for _ in range(NBUF)],
            *[pltpu.SemaphoreType.DMA for _ in range(2 * NBUF)],
        ],
    )
    def sc_embed(idx_hbm, table_hbm, out_hbm, idx_v, *bufs_and_sems):
        rows = bufs_and_sems[:NBUF]
        gsem = bufs_and_sems[NBUF : 2 * NBUF]
        ssem = bufs_and_sems[2 * NBUF : 3 * NBUF]

        wid = lax.axis_index("s") * num_cores + lax.axis_index("c")
        base = wid * b_per_w
        xr = wid // w_per_row
        xc = (wid % w_per_row) * b_per_w
        pltpu.sync_copy(idx_hbm.at[xr, pl.ds(xc, b_per_w)], idx_v)

        def issue_gather(off, L, k):
            off = pl.multiple_of(off, C0)
            return pltpu.async_copy(
                table_hbm.at[idx_v.at[pl.ds(off, L)]],
                rows[k].at[pl.ds(0, L)],
                gsem[k],
            )

        def issue_scatter(off, L, k):
            off = pl.multiple_of(base + off, C0)
            return pltpu.async_copy(
                rows[k].at[pl.ds(0, L)], out_hbm.at[pl.ds(off, L)], ssem[k]
            )

        def wait_gather(k, L=C):
            pltpu.make_async_copy(
                out_hbm.at[pl.ds(0, L)], rows[k].at[pl.ds(0, L)], gsem[k]
            ).wait()

        def wait_scatter(k, L=C):
            pltpu.make_async_copy(
                rows[k].at[pl.ds(0, L)], out_hbm.at[pl.ds(0, L)], ssem[k]
            ).wait()

        def scale_chunk(k, L=C):
            def row_body(r, cc, rv=rows[k]):
                for c in range(VECS_PER_ROW):
                    s = pl.ds(c * LANES, LANES)
                    rv[r, s] = rv[r, s] * SCALE
                return cc

            lax.fori_loop(0, L, row_body, 0)

        issue_gather(0, C0, 0)
        issue_gather(C0, C, 1)
        issue_gather(C0 + C, C, 2)

        wait_gather(0, C0)
        scale_chunk(0, C0)
        issue_scatter(0, C0, 0)

        def group(j, carry):
            for dg in range(1, NBUF + 1):
                g = j * NBUF + dg
                off = g * C - (C - C0)
                k = dg % NBUF
                kn = (k + 2) % NBUF
                wait_gather(k)
                scale_chunk(k)
                issue_scatter(off, C, k)
                if dg == 1:
                    @pl.when(j == 0)
                    def _():
                        wait_scatter(kn, C0)

                    @pl.when(j > 0)
                    def _():
                        wait_scatter(kn)

                    issue_gather(off + 2 * C, C, kn)
                elif dg == 2:
                    wait_scatter(kn)

                    @pl.when(j < n_groups - 1)
                    def _():
                        issue_gather(off + 2 * C, C, kn)

                    @pl.when(j == n_groups - 1)
                    def _():
                        issue_gather(chunk_off(n_full + 1), CT, kn)
                else:
                    wait_scatter(kn)

                    @pl.when(j < n_groups - 1)
                    def _():
                        issue_gather(off + 2 * C, C, kn)
            return carry

        lax.fori_loop(0, n_groups, group, 0)

        kt = (n_full + 1) % NBUF
        wait_gather(kt, CT)
        scale_chunk(kt, CT)
        issue_scatter(chunk_off(n_full + 1), CT, kt)
        wait_scatter(n_full % NBUF)
        wait_scatter(kt, CT)

    return sc_embed


def kernel(x, table):
    idx = x.astype(jnp.int32)
    out = _build_sc_embed(x.shape[0], x.shape[1], 2, 16)(idx, table)
    return out.reshape(x.shape[0], x.shape[1], D_MODEL)


# speedup vs baseline: 1.0195x; 1.0195x over previous
import functools
import math

import jax
import jax.numpy as jnp
from jax import lax
from jax.experimental import pallas as pl
from jax.experimental.pallas import tpu as pltpu
from jax.experimental.pallas import tpu_sc as plsc

D_MODEL = 1024
SCALE = math.sqrt(D_MODEL)
LANES = 16
VECS_PER_ROW = D_MODEL // LANES
C = 32
NBUF = 3


@functools.lru_cache(maxsize=None)
def _build_sc_embed(rows_x, cols_x, num_cores, num_subcores):
    B = rows_x * cols_x
    NW = num_cores * num_subcores
    b_per_w = B // NW
    w_per_row = cols_x // b_per_w
    assert w_per_row * b_per_w == cols_x
    C0 = 8
    n_full = (b_per_w - C0) // C
    CT = b_per_w - C0 - (n_full - 1) * C - C
    assert CT == C - C0
    n_groups = n_full // NBUF
    assert n_groups * NBUF == n_full and n_groups >= 2

    def chunk_off(g):
        return 0 if g == 0 else C0 + (g - 1) * C
    mesh = plsc.VectorSubcoreMesh(core_axis_name="c", subcore_axis_name="s")

    @functools.partial(
        pl.kernel,
        mesh=mesh,
        out_type=jax.ShapeDtypeStruct((B, D_MODEL), jnp.float32),
        scratch_types=[
            pltpu.VMEM((b_per_w,), jnp.int32),
            *[pltpu.VMEM((C, D_MODEL), jnp.float32) for _ in range(NBUF)],
            *[pltpu.SemaphoreType.DMA for _ in range(2 * NBUF)],
        ],
    )
    def sc_embed(idx_hbm, table_hbm, out_hbm, idx_v, *bufs_and_sems):
        rows = bufs_and_sems[:NBUF]
        gsem = bufs_and_sems[NBUF : 2 * NBUF]
        ssem = bufs_and_sems[2 * NBUF : 3 * NBUF]

        wid = lax.axis_index("s") * num_cores + lax.axis_index("c")
        base = wid * b_per_w
        xr = wid // w_per_row
        xc = (wid % w_per_row) * b_per_w
        pltpu.sync_copy(idx_hbm.at[xr, pl.ds(xc, b_per_w)], idx_v)

        def issue_gather(off, L, k):
            off = pl.multiple_of(off, C0)
            return pltpu.async_copy(
                table_hbm.at[idx_v.at[pl.ds(off, L)]],
                rows[k].at[pl.ds(0, L)],
                gsem[k],
            )

        def issue_scatter(off, L, k):
            off = pl.multiple_of(base + off, C0)
            return pltpu.async_copy(
                rows[k].at[pl.ds(0, L)], out_hbm.at[pl.ds(off, L)], ssem[k]
            )

        def wait_gather(k, L=C):
            pltpu.make_async_copy(
                out_hbm.at[pl.ds(0, L)], rows[k].at[pl.ds(0, L)], gsem[k]
            ).wait()

        def wait_scatter(k, L=C):
            pltpu.make_async_copy(
                rows[k].at[pl.ds(0, L)], out_hbm.at[pl.ds(0, L)], ssem[k]
            ).wait()

        def scale_chunk(k, L=C):
            def row_body(r, cc, rv=rows[k]):
                for c in range(VECS_PER_ROW):
                    s = pl.ds(c * LANES, LANES)
                    rv[r, s] = rv[r, s] * SCALE
                return cc

            lax.fori_loop(0, L, row_body, 0)

        issue_gather(0, C0, 0)
        issue_gather(C0, C, 1)

        wait_gather(0, C0)
        scale_chunk(0, C0)
        issue_scatter(0, C0, 0)
        issue_gather(C0 + C, C, 2)

        def group(j, carry):
            for dg in range(1, NBUF + 1):
                g = j * NBUF + dg
                off = g * C - (C - C0)
                k = dg % NBUF
                kn = (k + 2) % NBUF
                wait_gather(k)
                scale_chunk(k)
                issue_scatter(off, C, k)
                if dg == 1:
                    @pl.when(j == 0)
                    def _():
                        wait_scatter(kn, C0)

                    @pl.when(j > 0)
                    def _():
                        wait_scatter(kn)

                    issue_gather(off + 2 * C, C, kn)
                elif dg == 2:
                    wait_scatter(kn)

                    @pl.when(j < n_groups - 1)
                    def _():
                        issue_gather(off + 2 * C, C, kn)

                    @pl.when(j == n_groups - 1)
                    def _():
                        issue_gather(chunk_off(n_full + 1), CT, kn)
                else:
                    wait_scatter(kn)

                    @pl.when(j < n_groups - 1)
                    def _():
                        issue_gather(off + 2 * C, C, kn)
            return carry

        lax.fori_loop(0, n_groups, group, 0)

        kt = (n_full + 1) % NBUF
        wait_gather(kt, CT)
        scale_chunk(kt, CT)
        issue_scatter(chunk_off(n_full + 1), CT, kt)
        wait_scatter(n_full % NBUF)
        wait_scatter(kt, CT)

    return sc_embed


def kernel(x, table):
    idx = x.astype(jnp.int32)
    out = _build_sc_embed(x.shape[0], x.shape[1], 2, 16)(idx, table)
    return out.reshape(x.shape[0], x.shape[1], D_MODEL)
